# Initial kernel scaffold; baseline (speedup 1.0000x reference)
import jax
import jax.numpy as jnp
from jax.experimental import pallas as pl


def kernel(x, hyperedge_index, W_in, b_in, g_in, bn_in, Wc, bc, Wq, bq, Wk, bk, Wv, bv, Wo, bo, g1, b1, g2, b2, Wf1, bf1, Wf2, bf2, Wd1, bd1, Wd2, bd2, Wd3, bd3, Wr1, br1, Wr2, br2, Wr3, br3, Wt1, bt1, Wt2, bt2, Wt3, bt3):
    raise NotImplementedError("write your pallas kernel here")


# speedup vs baseline: 10.1206x; 10.1206x over previous
import math

import jax
import jax.numpy as jnp
from jax.experimental import pallas as pl

N = 200
NODE_DIM = 8
HID = 128
L = 6
HEADS = 4
HD = HID // HEADS
HE = 512
K = 8
FLAT = HE * K


def _mm(a, b):
    return jax.lax.dot_general(a, b, (((1,), (0,)), ((), ())),
                               preferred_element_type=jnp.float32)


def _mmh(a, b):
    return jax.lax.dot_general(a, b, (((1,), (0,)), ((), ())),
                               precision=jax.lax.Precision.HIGHEST,
                               preferred_element_type=jnp.float32)


def _mtm(a, b):
    return jax.lax.dot_general(a, b, (((0,), (0,)), ((), ())),
                               precision=jax.lax.Precision.HIGHEST,
                               preferred_element_type=jnp.float32)


def _b16(a):
    return a.astype(jnp.bfloat16).astype(jnp.float32)


def _ln(x, g, b):
    mu = x.mean(-1, keepdims=True)
    v = ((x - mu) ** 2).mean(-1, keepdims=True)
    return g * (x - mu) / jnp.sqrt(v + 1e-5) + b


def _pe_table():
    pos = jnp.arange(N, dtype=jnp.float32)[:, None]
    div = jnp.exp(jnp.arange(0, HID, 2, dtype=jnp.float32)
                  * (-math.log(10000.0) / HID))
    pe = jnp.zeros((N, HID), dtype=jnp.float32)
    pe = pe.at[:, 0::2].set(jnp.sin(pos * div))
    pe = pe.at[:, 1::2].set(jnp.cos(pos * div))
    return pe


def _prep_kernel(idx_ref, G_ref, S_ref, invc_ref):
    idx0 = idx_ref[:, 0:1]
    node_iota = jax.lax.broadcasted_iota(jnp.int32, (FLAT, N), 1)
    G = (idx0 == node_iota).astype(jnp.float32)
    G_ref[...] = G
    S_ref[...] = jnp.sum(G.reshape(HE, K, N), axis=1)
    cnt = _mtm(G, jnp.ones((FLAT, 1), jnp.float32))
    invc_ref[...] = 1.0 / jnp.maximum(cnt, 1.0)


def _inproj_kernel(x_ref, pe_ref, W_in_ref, b_in_ref, g_in_ref, bn_in_ref,
                   h_ref):
    h = _ln(_mm(x_ref[...], W_in_ref[...]) + b_in_ref[...],
            g_in_ref[...], bn_in_ref[...])
    h_ref[...] = jax.nn.gelu(h) + pe_ref[...]


def _layer_kernel(h_ref, G_ref, S_ref, invc_ref,
                  Wc_ref, bc_ref, Wq_ref, bq_ref, Wk_ref, bk_ref,
                  Wv_ref, bv_ref, Wo_ref, bo_ref,
                  g1_ref, b1_ref, g2_ref, b2_ref,
                  Wf1_ref, bf1_ref, Wf2_ref, bf2_ref,
                  ho_ref):
    f32 = jnp.float32
    G = G_ref[...]
    S = S_ref[...]
    inv_cnt = invc_ref[...]
    h = h_ref[...]

    ha = jax.lax.broadcasted_iota(jnp.int32, (HID, HID), 0) // HD
    hb = jax.lax.broadcasted_iota(jnp.int32, (HID, HID), 1) // HD
    Bmask = (ha == hb).astype(f32)

    xt = _mm(h, Wc_ref[...]) + bc_ref[...]
    eh = _mmh(S, xt) * (1.0 / K)
    conv = _mtm(S, eh) * inv_cnt
    h = _ln(h + conv, g1_ref[...], b1_ref[...])

    Q = _mm(h, Wq_ref[...]) + bq_ref[...]
    Kp = _mm(h, Wk_ref[...]) + bk_ref[...]
    V = _mm(h, Wv_ref[...]) + bv_ref[...]
    qg = _mmh(G, Q)
    kg = _mmh(G, Kp)
    vg = _mmh(G, V)
    kg3 = kg.reshape(HE, K, HID)
    vg3 = vg.reshape(HE, K, HID)

    scale = 1.0 / math.sqrt(HD)
    qb = _b16(qg)
    s_list = []
    for rot in range(K):
        if rot == 0:
            krot = kg
        else:
            krot = jnp.concatenate(
                [kg3[:, rot:, :], kg3[:, :rot, :]], axis=1
            ).reshape(FLAT, HID)
        s_list.append(_mmh(qb * _b16(krot), Bmask) * scale)
    m = s_list[0]
    for rot in range(1, K):
        m = jnp.maximum(m, s_list[rot])
    e_list = [jnp.exp(s - m) for s in s_list]
    z = e_list[0]
    for rot in range(1, K):
        z = z + e_list[rot]
    acc = _b16(e_list[0] / z) * _b16(vg)
    for rot in range(1, K):
        vrot = jnp.concatenate(
            [vg3[:, rot:, :], vg3[:, :rot, :]], axis=1
        ).reshape(FLAT, HID)
        acc = acc + _b16(e_list[rot] / z) * _b16(vrot)
    att = acc

    out = _mtm(G, att) * inv_cnt
    h = h + _mm(out, Wo_ref[...]) + bo_ref[...]

    ff = _mm(jax.nn.gelu(_mm(h, Wf1_ref[...]) + bf1_ref[...]),
             Wf2_ref[...]) + bf2_ref[...]
    ho_ref[...] = _ln(h + ff, g2_ref[...], b2_ref[...])


def _heads_kernel(h_ref,
                  Wd1_ref, bd1_ref, Wd2_ref, bd2_ref, Wd3_ref, bd3_ref,
                  Wr1_ref, br1_ref, Wr2_ref, br2_ref, Wr3_ref, br3_ref,
                  Wt1_ref, bt1_ref, Wt2_ref, bt2_ref, Wt3_ref, bt3_ref,
                  d_ref, r_ref, t_ref):
    h = h_ref[...]
    pooled = jnp.concatenate(
        [jnp.mean(h, axis=0, keepdims=True),
         jnp.max(h, axis=0, keepdims=True),
         jnp.sum(h, axis=0, keepdims=True)], axis=1)

    def head(A1, a1, A2, a2, A3, a3):
        zz = jax.nn.gelu(_mm(pooled, A1) + a1)
        zz = jax.nn.gelu(_mm(zz, A2) + a2)
        return _mm(zz, A3) + a3

    d_ref[...] = head(Wd1_ref[...], bd1_ref[...], Wd2_ref[...],
                      bd2_ref[...], Wd3_ref[...], bd3_ref[...])
    r_ref[...] = jax.nn.sigmoid(head(Wr1_ref[...], br1_ref[...],
                                     Wr2_ref[...], br2_ref[...],
                                     Wr3_ref[...], br3_ref[...]))
    t_ref[...] = head(Wt1_ref[...], bt1_ref[...], Wt2_ref[...],
                      bt2_ref[...], Wt3_ref[...], bt3_ref[...])


def kernel(x, hyperedge_index, W_in, b_in, g_in, bn_in, Wc, bc, Wq, bq,
           Wk, bk, Wv, bv, Wo, bo, g1, b1, g2, b2, Wf1, bf1, Wf2, bf2,
           Wd1, bd1, Wd2, bd2, Wd3, bd3, Wr1, br1, Wr2, br2, Wr3, br3,
           Wt1, bt1, Wt2, bt2, Wt3, bt3):
    f32 = jnp.float32
    flat = hyperedge_index.astype(jnp.int32).reshape(FLAT, 1)
    row = lambda v: v.reshape(1, -1)

    G, S, inv_cnt = pl.pallas_call(
        _prep_kernel,
        out_shape=[jax.ShapeDtypeStruct((FLAT, N), f32),
                   jax.ShapeDtypeStruct((HE, N), f32),
                   jax.ShapeDtypeStruct((N, 1), f32)],
    )(flat)

    h = pl.pallas_call(
        _inproj_kernel,
        out_shape=jax.ShapeDtypeStruct((N, HID), f32),
    )(x, _pe_table(), W_in, row(b_in), row(g_in), row(bn_in))

    layer_call = pl.pallas_call(
        _layer_kernel,
        out_shape=jax.ShapeDtypeStruct((N, HID), f32),
    )
    for l in range(L):
        h = layer_call(h, G, S, inv_cnt,
                       Wc[l], row(bc[l]), Wq[l], row(bq[l]),
                       Wk[l], row(bk[l]), Wv[l], row(bv[l]),
                       Wo[l], row(bo[l]),
                       row(g1[l]), row(b1[l]), row(g2[l]), row(b2[l]),
                       Wf1[l], row(bf1[l]), Wf2[l], row(bf2[l]))

    d, r, t = pl.pallas_call(
        _heads_kernel,
        out_shape=[jax.ShapeDtypeStruct((1, 1), f32)] * 3,
    )(h,
      Wd1, row(bd1), Wd2, row(bd2), Wd3, row(bd3),
      Wr1, row(br1), Wr2, row(br2), Wr3, row(br3),
      Wt1, row(bt1), Wt2, row(bt2), Wt3, row(bt3))
    return d.reshape(1), r.reshape(1), t.reshape(1)
